# baseline (speedup 1.0000x reference)
import functools

import jax
import jax.numpy as jnp
from jax import lax
from jax.experimental import pallas as pl
from jax.experimental.pallas import tpu as pltpu

_LANE = 128
_TM = 128


def _fused_lora_kernel(x_ref, w_ref, at_ref, b_ref, bias_ref, o_ref):
    xb = x_ref[...].astype(jnp.bfloat16)
    nt = (((1,), (1,)), ((), ()))

    xa = lax.dot_general(xb, at_ref[...], (((1,), (0,)), ((), ())),
                         preferred_element_type=jnp.float32)

    acc = lax.dot_general(xb, w_ref[...], nt,
                          preferred_element_type=jnp.float32)

    lora = lax.dot_general(xa.astype(jnp.bfloat16), b_ref[...], nt,
                           preferred_element_type=jnp.float32)
    o_ref[...] = acc + lora + bias_ref[...]


def kernel(x, w, bias, A, B):
    lead = x.shape[:-1]
    K = x.shape[-1]
    N = w.shape[0]
    r = A.shape[0]
    rp = -(-r // _LANE) * _LANE

    x2 = x.reshape(-1, K)
    M = x2.shape[0]
    tm = min(_TM, M)

    wb = w.astype(jnp.bfloat16)
    at = jnp.pad(A.T.astype(jnp.bfloat16), ((0, 0), (0, rp - r)))
    bb = jnp.pad(B.astype(jnp.bfloat16), ((0, 0), (0, rp - r)))
    bias2 = bias.astype(jnp.float32).reshape(1, N)

    y = pl.pallas_call(
        _fused_lora_kernel,
        out_shape=jax.ShapeDtypeStruct((M, N), jnp.float32),
        grid=(M // tm,),
        in_specs=[
            pl.BlockSpec((tm, K), lambda i: (i, 0)),
            pl.BlockSpec((N, K), lambda i: (0, 0)),
            pl.BlockSpec((K, rp), lambda i: (0, 0)),
            pl.BlockSpec((N, rp), lambda i: (0, 0)),
            pl.BlockSpec((1, N), lambda i: (0, 0)),
        ],
        out_specs=pl.BlockSpec((tm, N), lambda i: (i, 0)),
        compiler_params=pltpu.CompilerParams(
            dimension_semantics=("parallel",),
            vmem_limit_bytes=56 * 1024 * 1024,
        ),
        cost_estimate=pl.CostEstimate(
            flops=2 * M * K * N + 2 * M * K * rp + 2 * M * rp * N,
            transcendentals=0,
            bytes_accessed=(M * K + M * N) * 4 + (N * K + K * rp + N * rp) * 2,
        ),
    )(x2, wb, at, bb, bias2)
    return y.reshape(*lead, N)


# speedup vs baseline: 1.8616x; 1.8616x over previous
import jax
import jax.numpy as jnp
from jax import lax
from jax.experimental import pallas as pl
from jax.experimental.pallas import tpu as pltpu

_TM = 256


def _fused_lora_kernel(x_ref, w_ref, a_ref, b_ref, bias_ref, o_ref, wb_ref):
    @pl.when(pl.program_id(0) == 0)
    def _():
        wb_ref[...] = w_ref[...].astype(jnp.bfloat16)

    nt = (((1,), (1,)), ((), ()))
    xb = x_ref[...].astype(jnp.bfloat16)

    xa = lax.dot_general(xb, a_ref[...].astype(jnp.bfloat16), nt,
                         preferred_element_type=jnp.float32)

    acc = lax.dot_general(xb, wb_ref[...], nt,
                          preferred_element_type=jnp.float32)

    lora = lax.dot_general(xa.astype(jnp.bfloat16),
                           b_ref[...].astype(jnp.bfloat16), nt,
                           preferred_element_type=jnp.float32)
    o_ref[...] = acc + lora + bias_ref[...]


def kernel(x, w, bias, A, B):
    lead = x.shape[:-1]
    K = x.shape[-1]
    N = w.shape[0]
    r = A.shape[0]

    x2 = x.reshape(-1, K)
    M = x2.shape[0]
    tm = min(_TM, M)
    bias2 = bias.reshape(1, N)

    y = pl.pallas_call(
        _fused_lora_kernel,
        out_shape=jax.ShapeDtypeStruct((M, N), jnp.float32),
        grid=(M // tm,),
        in_specs=[
            pl.BlockSpec((tm, K), lambda i: (i, 0)),
            pl.BlockSpec((N, K), lambda i: (0, 0)),
            pl.BlockSpec((r, K), lambda i: (0, 0)),
            pl.BlockSpec((N, r), lambda i: (0, 0)),
            pl.BlockSpec((1, N), lambda i: (0, 0)),
        ],
        out_specs=pl.BlockSpec((tm, N), lambda i: (i, 0)),
        scratch_shapes=[pltpu.VMEM((N, K), jnp.bfloat16)],
        compiler_params=pltpu.CompilerParams(
            dimension_semantics=("arbitrary",),
            vmem_limit_bytes=60 * 1024 * 1024,
        ),
        cost_estimate=pl.CostEstimate(
            flops=2 * M * K * N + 2 * M * K * r + 2 * M * r * N,
            transcendentals=0,
            bytes_accessed=(M * K + M * N + N * K) * 4,
        ),
    )(x2, w, A, B, bias2)
    return y.reshape(*lead, N)


# speedup vs baseline: 1.9083x; 1.0251x over previous
import jax
import jax.numpy as jnp
from jax import lax
from jax.experimental import pallas as pl
from jax.experimental.pallas import tpu as pltpu

_TM = 512


def _fused_lora_kernel(x_ref, w_ref, a_ref, b_ref, bias_ref, o_ref, wb_ref):
    @pl.when(pl.program_id(0) == 0)
    def _():
        wb_ref[...] = w_ref[...].astype(jnp.bfloat16)

    nt = (((1,), (1,)), ((), ()))
    xb = x_ref[...].astype(jnp.bfloat16)

    xa = lax.dot_general(xb, a_ref[...].astype(jnp.bfloat16), nt,
                         preferred_element_type=jnp.float32)

    acc = lax.dot_general(xb, wb_ref[...], nt,
                          preferred_element_type=jnp.float32)

    lora = lax.dot_general(xa.astype(jnp.bfloat16),
                           b_ref[...].astype(jnp.bfloat16), nt,
                           preferred_element_type=jnp.float32)
    o_ref[...] = acc + lora + bias_ref[...]


def kernel(x, w, bias, A, B):
    lead = x.shape[:-1]
    K = x.shape[-1]
    N = w.shape[0]
    r = A.shape[0]

    x2 = x.reshape(-1, K)
    M = x2.shape[0]
    tm = min(_TM, M)
    bias2 = bias.reshape(1, N)

    y = pl.pallas_call(
        _fused_lora_kernel,
        out_shape=jax.ShapeDtypeStruct((M, N), jnp.float32),
        grid=(M // tm,),
        in_specs=[
            pl.BlockSpec((tm, K), lambda i: (i, 0)),
            pl.BlockSpec((N, K), lambda i: (0, 0)),
            pl.BlockSpec((r, K), lambda i: (0, 0)),
            pl.BlockSpec((N, r), lambda i: (0, 0)),
            pl.BlockSpec((1, N), lambda i: (0, 0)),
        ],
        out_specs=pl.BlockSpec((tm, N), lambda i: (i, 0)),
        scratch_shapes=[pltpu.VMEM((N, K), jnp.bfloat16)],
        compiler_params=pltpu.CompilerParams(
            dimension_semantics=("arbitrary",),
            vmem_limit_bytes=60 * 1024 * 1024,
        ),
        cost_estimate=pl.CostEstimate(
            flops=2 * M * K * N + 2 * M * K * r + 2 * M * r * N,
            transcendentals=0,
            bytes_accessed=(M * K + M * N + N * K) * 4,
        ),
    )(x2, w, A, B, bias2)
    return y.reshape(*lead, N)


# speedup vs baseline: 2.1300x; 1.1162x over previous
import jax
import jax.numpy as jnp
from jax import lax
from jax.experimental import pallas as pl
from jax.experimental.pallas import tpu as pltpu

_TM = 512
_RP = 128


def _fused_lora_kernel(x_ref, w_ref, a_ref, b_ref, bias_ref, o_ref,
                       wcat_ref, xb_ref):
    N, K = w_ref.shape
    r = a_ref.shape[0]

    @pl.when(pl.program_id(0) == 0)
    def _():
        wcat_ref[:N, :] = w_ref[...].astype(jnp.bfloat16)
        wcat_ref[N:N + r, :] = a_ref[...].astype(jnp.bfloat16)
        wcat_ref[N + r:, :] = jnp.zeros((_RP - r, K), jnp.bfloat16)

    xb_ref[...] = x_ref[...].astype(jnp.bfloat16)

    nt = (((1,), (1,)), ((), ()))
    big = lax.dot_general(xb_ref[...], wcat_ref[...], nt,
                          preferred_element_type=jnp.float32)

    lora = lax.dot_general(big[:, N:N + r].astype(jnp.bfloat16),
                           b_ref[...].astype(jnp.bfloat16), nt,
                           preferred_element_type=jnp.float32)
    o_ref[...] = big[:, :N] + lora + bias_ref[...]


def kernel(x, w, bias, A, B):
    lead = x.shape[:-1]
    K = x.shape[-1]
    N = w.shape[0]
    r = A.shape[0]

    x2 = x.reshape(-1, K)
    M = x2.shape[0]
    tm = min(_TM, M)
    bias2 = bias.reshape(1, N)

    y = pl.pallas_call(
        _fused_lora_kernel,
        out_shape=jax.ShapeDtypeStruct((M, N), jnp.float32),
        grid=(M // tm,),
        in_specs=[
            pl.BlockSpec((tm, K), lambda i: (i, 0)),
            pl.BlockSpec((N, K), lambda i: (0, 0)),
            pl.BlockSpec((r, K), lambda i: (0, 0)),
            pl.BlockSpec((N, r), lambda i: (0, 0)),
            pl.BlockSpec((1, N), lambda i: (0, 0)),
        ],
        out_specs=pl.BlockSpec((tm, N), lambda i: (i, 0)),
        scratch_shapes=[
            pltpu.VMEM((N + _RP, K), jnp.bfloat16),
            pltpu.VMEM((tm, K), jnp.bfloat16),
        ],
        compiler_params=pltpu.CompilerParams(
            dimension_semantics=("arbitrary",),
            vmem_limit_bytes=60 * 1024 * 1024,
        ),
        cost_estimate=pl.CostEstimate(
            flops=2 * M * K * (N + _RP) + 2 * M * r * N,
            transcendentals=0,
            bytes_accessed=(M * K + M * N + N * K) * 4,
        ),
    )(x2, w, A, B, bias2)
    return y.reshape(*lead, N)
